# Initial kernel scaffold; baseline (speedup 1.0000x reference)
import jax
import jax.numpy as jnp
from jax.experimental import pallas as pl


def kernel(x_ids, table):
    raise NotImplementedError("write your pallas kernel here")


# speedup vs baseline: 4.4996x; 4.4996x over previous
import math
import functools

import jax
import jax.numpy as jnp
from jax.experimental import pallas as pl
from jax.experimental.pallas import tpu as pltpu


def _round_up(x, m):
    return (x + m - 1) // m * m


def _gather_scale_kernel(ids_ref, table_hbm, out_ref, sem, *, tile, scale):
    V = table_hbm.shape[0]
    base = pl.program_id(0) * tile

    @pl.loop(0, tile)
    def _(t):
        row = ids_ref[base + t]
        row = jnp.minimum(jnp.maximum(row, 0), V - 1)
        pltpu.make_async_copy(
            table_hbm.at[pl.ds(row, 1), :],
            out_ref.at[pl.ds(t, 1), :],
            sem,
        ).start()

    pltpu.make_async_copy(
        table_hbm.at[pl.ds(0, tile), :],
        out_ref.at[pl.ds(0, tile), :],
        sem,
    ).wait()

    out_ref[...] = out_ref[...] * jnp.float32(scale)


def kernel(x_ids, table):
    B, S = x_ids.shape
    V, D = table.shape
    N = B * S
    scale = math.sqrt(D)

    tile = min(512, _round_up(N, 8))
    if _round_up(N, tile) // tile < 2 and N > 8:
        tile = min(tile, _round_up((N + 1) // 2, 8))
    n_pad = _round_up(N, tile)

    flat_ids = x_ids.reshape(N).astype(jnp.int32)
    if n_pad != N:
        flat_ids = jnp.pad(flat_ids, (0, n_pad - N))

    itemsize = jnp.dtype(table.dtype).itemsize
    vmem_limit = int(min(4 * tile * D * itemsize + (8 << 20), 56 << 20))

    grid_spec = pltpu.PrefetchScalarGridSpec(
        num_scalar_prefetch=1,
        grid=(n_pad // tile,),
        in_specs=[pl.BlockSpec(memory_space=pl.ANY)],
        out_specs=pl.BlockSpec((tile, D), lambda i, ids: (i, 0)),
        scratch_shapes=[pltpu.SemaphoreType.DMA],
    )
    out_flat = pl.pallas_call(
        functools.partial(_gather_scale_kernel, tile=tile, scale=scale),
        out_shape=jax.ShapeDtypeStruct((n_pad, D), table.dtype),
        grid_spec=grid_spec,
        compiler_params=pltpu.CompilerParams(
            dimension_semantics=("parallel",),
            vmem_limit_bytes=vmem_limit,
            disable_bounds_checks=True,
        ),
        name="embedding_gather_scale",
    )(flat_ids, table)

    return out_flat[:N].reshape(B, S, D)


# speedup vs baseline: 4.7246x; 1.0500x over previous
import math
import functools

import jax
import jax.numpy as jnp
from jax.experimental import pallas as pl
from jax.experimental.pallas import tpu as pltpu


def _round_up(x, m):
    return (x + m - 1) // m * m


def _gather_scale_kernel(ids_ref, table_hbm, out_ref, sem0, sem1, *, tile,
                         scale):
    V = table_hbm.shape[0]
    base = pl.program_id(0) * tile

    @pl.loop(0, tile // 2)
    def _(tq):
        for u, sem, prio in ((0, sem0, 0), (1, sem1, 1)):
            t = tq * 2 + u
            row = ids_ref[base + t]
            row = jnp.minimum(jnp.maximum(row, 0), V - 1)
            pltpu.async_copy(
                table_hbm.at[pl.ds(row, 1), :],
                out_ref.at[pl.ds(t, 1), :],
                sem,
                priority=prio,
            )

    half = tile // 2
    pltpu.make_async_copy(
        table_hbm.at[pl.ds(0, half), :],
        out_ref.at[pl.ds(0, half), :],
        sem0,
    ).wait()
    pltpu.make_async_copy(
        table_hbm.at[pl.ds(0, half), :],
        out_ref.at[pl.ds(0, half), :],
        sem1,
    ).wait()

    out_ref[...] = out_ref[...] * jnp.float32(scale)


def kernel(x_ids, table):
    B, S = x_ids.shape
    V, D = table.shape
    N = B * S
    scale = math.sqrt(D)

    tile = min(512, _round_up(N, 8))
    if _round_up(N, tile) // tile < 2 and N > 8:
        tile = min(tile, _round_up((N + 1) // 2, 8))
    n_pad = _round_up(N, tile)

    flat_ids = x_ids.reshape(N).astype(jnp.int32)
    if n_pad != N:
        flat_ids = jnp.pad(flat_ids, (0, n_pad - N))

    itemsize = jnp.dtype(table.dtype).itemsize
    vmem_limit = int(min(4 * tile * D * itemsize + (8 << 20), 56 << 20))

    grid_spec = pltpu.PrefetchScalarGridSpec(
        num_scalar_prefetch=1,
        grid=(n_pad // tile,),
        in_specs=[pl.BlockSpec(memory_space=pl.ANY)],
        out_specs=pl.BlockSpec((tile, D), lambda i, ids: (i, 0)),
        scratch_shapes=[pltpu.SemaphoreType.DMA, pltpu.SemaphoreType.DMA],
    )
    out_flat = pl.pallas_call(
        functools.partial(_gather_scale_kernel, tile=tile, scale=scale),
        out_shape=jax.ShapeDtypeStruct((n_pad, D), table.dtype),
        grid_spec=grid_spec,
        compiler_params=pltpu.CompilerParams(
            dimension_semantics=("parallel",),
            vmem_limit_bytes=vmem_limit,
            disable_bounds_checks=True,
        ),
        name="embedding_gather_scale",
    )(flat_ids, table)

    return out_flat[:N].reshape(B, S, D)
